# Initial kernel scaffold; baseline (speedup 1.0000x reference)
import jax
import jax.numpy as jnp
from jax.experimental import pallas as pl


def kernel(node_features, edge_features, coords, edge_index, params):
    raise NotImplementedError("write your pallas kernel here")


# speedup vs baseline: 1.8595x; 1.8595x over previous
import functools

import jax
import jax.numpy as jnp
from jax import lax
from jax.experimental import pallas as pl
from jax.experimental.pallas import tpu as pltpu
from jax.experimental.pallas import tpu_sc as plsc

N_NODES = 10000
N_EDGES = 160000
NODE_DIM = 64
EDGE_DIM = 64
HIDDEN = 128
N_LAYERS = 12
ALPHA = 0.1
TEMP = 10.0

TW = 80
MW = 144

NC = 2
NS = 16
NW = NC * NS
CHUNK = 128
N_CHUNKS = N_EDGES // CHUNK
NPT = 313
NPT_LAST = N_NODES // NC - (NS - 1) * NPT
ACC_ROWS = NS * NPT
TRASH_ROW = N_NODES // NC

BLK_E = 4000
BLK_N = 2000

_f32 = jnp.float32
_HI = lax.Precision.DEFAULT


@functools.lru_cache(maxsize=None)
def _sc_kernels():
    mesh = plsc.VectorSubcoreMesh(core_axis_name="c", subcore_axis_name="s",
                                  num_cores=NC, num_subcores=NS)

    @functools.partial(
        pl.kernel,
        out_type=(jax.ShapeDtypeStruct((N_EDGES, TW), _f32),
                  jax.ShapeDtypeStruct((N_EDGES, TW), _f32)),
        mesh=mesh,
        scratch_types=[pltpu.VMEM((CHUNK,), jnp.int32),
                       pltpu.VMEM((CHUNK,), jnp.int32),
                       pltpu.VMEM((CHUNK, TW), _f32),
                       pltpu.VMEM((CHUNK, TW), _f32),
                       pltpu.SemaphoreType.DMA,
                       pltpu.SemaphoreType.DMA],
        compiler_params=pltpu.CompilerParams(use_tc_tiling_on_sc=False),
    )
    def sc_gather(table, row, col, grow, gcol, idx_r, idx_c, buf_r, buf_c,
                  sem_r, sem_c):
        wid = lax.axis_index("s") * NC + lax.axis_index("c")
        extra = N_CHUNKS % NW
        nloop = jnp.where(wid < extra, N_CHUNKS // NW + 1, N_CHUNKS // NW)

        def body(i, carry):
            base = (wid + i * NW) * CHUNK
            pltpu.sync_copy(row.at[pl.ds(base, CHUNK)], idx_r)
            pltpu.sync_copy(col.at[pl.ds(base, CHUNK)], idx_c)
            cp_r = pltpu.async_copy(table.at[idx_r], buf_r, sem_r)
            cp_c = pltpu.async_copy(table.at[idx_c], buf_c, sem_c)
            cp_r.wait()
            cp_c.wait()
            pltpu.sync_copy(buf_r, grow.at[pl.ds(base, CHUNK)])
            pltpu.sync_copy(buf_c, gcol.at[pl.ds(base, CHUNK)])
            return carry

        lax.fori_loop(0, nloop, body, 0)

    @functools.partial(
        pl.kernel,
        out_type=jax.ShapeDtypeStruct((N_NODES, MW), _f32),
        mesh=mesh,
        scratch_types=[pltpu.VMEM((CHUNK,), jnp.int32),
                       pltpu.VMEM((CHUNK,), jnp.int32),
                       pltpu.VMEM((16,), jnp.int32),
                       pltpu.VMEM((CHUNK, MW), _f32),
                       pltpu.VMEM_SHARED((ACC_ROWS, MW), _f32),
                       pltpu.SemaphoreType.DMA],
        compiler_params=pltpu.CompilerParams(use_tc_tiling_on_sc=False, needs_layout_passes=False),
    )
    def sc_scatter(msc, perm, row_sorted, offs_rep, zeros_acc, out,
                   idx_v, pidx_v, off_v, buf, acc, sem):
        cid = lax.axis_index("c")
        sid = lax.axis_index("s")
        tid = cid * NS + sid
        zbase = sid * NPT
        pltpu.sync_copy(zeros_acc.at[pl.ds(zbase, NPT)],
                        acc.at[pl.ds(zbase, NPT)])
        plsc.subcore_barrier()

        pltpu.sync_copy(offs_rep.at[tid], off_v)
        o_lo = jnp.max(off_v[...])
        pltpu.sync_copy(offs_rep.at[tid + 1], off_v)
        o_hi = jnp.max(off_v[...])
        core_base = cid * (N_NODES // NC)
        lane = lax.iota(jnp.int32, 16)

        def cond(p):
            return p < o_hi

        def body(p):
            w0 = jnp.minimum((p // 8) * 8, N_EDGES - CHUNK)
            pltpu.sync_copy(row_sorted.at[pl.ds(w0, CHUNK)], idx_v)
            pltpu.sync_copy(perm.at[pl.ds(w0, CHUNK)], pidx_v)
            pltpu.async_copy(msc.at[pidx_v], buf, sem).wait()
            r_tail = idx_v[pl.ds(CHUNK - 16, 16)]
            last_row = jnp.max(jnp.where(lane == 15, r_tail, jnp.int32(-1)))
            c_last = jnp.int32(0)
            for j in range(CHUNK // 16):
                rj = idx_v[pl.ds(j * 16, 16)]
                c_last = c_last + jnp.sum(
                    jnp.where(rj == last_row, jnp.int32(1), jnp.int32(0)))
            ss = w0 + CHUNK - c_last
            wend = w0 + CHUNK
            keep_end = jnp.where(wend >= o_hi, o_hi,
                                 jnp.where(ss <= p, wend, ss))
            for j in range(CHUNK // 16):
                rj = idx_v[pl.ds(j * 16, 16)]
                pos = w0 + j * 16 + lane
                valid = (pos >= p) & (pos < keep_end)
                idx_v[pl.ds(j * 16, 16)] = jnp.where(
                    valid, rj - core_base, TRASH_ROW)
            pltpu.sync_copy(buf, acc.at[idx_v], add=True)
            return keep_end

        lax.while_loop(cond, body, o_lo)
        plsc.subcore_barrier()

        obase = core_base + sid * NPT
        @pl.when(sid < NS - 1)
        def _():
            pltpu.sync_copy(acc.at[pl.ds(zbase, NPT)],
                            out.at[pl.ds(obase, NPT)])
        @pl.when(sid == NS - 1)
        def _():
            pltpu.sync_copy(acc.at[pl.ds(zbase, NPT_LAST)],
                            out.at[pl.ds(obase, NPT_LAST)])

    return sc_gather, sc_scatter


def _ln(x, g, b):
    m = jnp.mean(x, axis=-1, keepdims=True)
    v = jnp.mean((x - m) ** 2, axis=-1, keepdims=True)
    return (x - m) / jnp.sqrt(v + 1e-5) * g + b


def _silu(x):
    return x * jax.nn.sigmoid(x)


def _full(shape):
    return pl.BlockSpec(shape, lambda i: tuple(0 for _ in shape))


def _embed_nodes_body(nf_ref, xc_ref, w_ref, b_ref, t_ref):
    h = jnp.dot(nf_ref[...], w_ref[...], preferred_element_type=_f32, precision=_HI)
    h = h + b_ref[...]
    t_ref[...] = jnp.concatenate([h, xc_ref[...]], axis=1)


def _embed_edges_body(ef_ref, w_ref, b_ref, e_ref):
    e = jnp.dot(ef_ref[...], w_ref[...], preferred_element_type=_f32, precision=_HI)
    e_ref[...] = e + b_ref[...]


def _edge_body(grow_ref, gcol_ref, e_ref,
               m0w_ref, b0_ref,
               mlng_ref, mlnb_ref, m1w_ref, m1b_ref, m2w_ref, m2b_ref,
               c0w_ref, c0b_ref, c1p_ref,
               e0w_ref, e0bias_ref, elng_ref, elnb_ref,
               e1w_ref, e1b_ref, eng_ref, enb_ref,
               msc_ref, enew_ref):
    grow = grow_ref[...]
    gcol = gcol_ref[...]
    e = e_ref[...]
    hr = grow[:, :NODE_DIM]
    hc = gcol[:, :NODE_DIM]
    diff16 = gcol[:, NODE_DIM:] - grow[:, NODE_DIM:]
    d2 = jnp.sum(diff16 * diff16, axis=-1, keepdims=True)
    dist = jnp.sqrt(d2)

    msg_in = jnp.concatenate([hr, hc, e, dist], axis=1)
    pre = jnp.dot(msg_in, m0w_ref[...],
                  preferred_element_type=_f32, precision=_HI) + b0_ref[...]
    m = _silu(pre)
    m = _ln(m, mlng_ref[...], mlnb_ref[...])
    m = _silu(jnp.dot(m, m1w_ref[...], preferred_element_type=_f32, precision=_HI)
              + m1b_ref[...])
    m = jnp.dot(m, m2w_ref[...], preferred_element_type=_f32, precision=_HI) + m2b_ref[...]

    t = _silu(jnp.dot(m, c0w_ref[...], preferred_element_type=_f32, precision=_HI)
              + c0b_ref[...])
    cw = jnp.dot(t, c1p_ref[...], preferred_element_type=_f32,
                 precision=_HI)[:, :1]
    cw = jnp.tanh(cw / TEMP)
    cwdir = cw * (diff16 / (dist + 1e-8))
    msc_ref[...] = jnp.concatenate([m, cwdir], axis=1)

    ee_in = jnp.concatenate([e, m], axis=1)
    ee = _silu(jnp.dot(ee_in, e0w_ref[...], preferred_element_type=_f32, precision=_HI)
               + e0bias_ref[...])
    ee = _ln(ee, elng_ref[...], elnb_ref[...])
    ee = jnp.dot(ee, e1w_ref[...], preferred_element_type=_f32, precision=_HI) + e1b_ref[...]
    enew_ref[...] = _ln(e + ee, eng_ref[...], enb_ref[...])


def _node_body(t_ref, agg_ref,
               n0w_ref, n0bias_ref, nlng_ref, nlnb_ref,
               n1w_ref, n1b_ref, nng_ref, nnb_ref, tnew_ref):
    t = t_ref[...]
    h = t[:, :NODE_DIM]
    x16 = t[:, NODE_DIM:]
    agg = agg_ref[...]
    h_agg = agg[:, :HIDDEN]
    x_agg16 = agg[:, HIDDEN:]
    nh_in = jnp.concatenate([h, h_agg], axis=1)
    nh = _silu(jnp.dot(nh_in, n0w_ref[...], preferred_element_type=_f32, precision=_HI)
               + n0bias_ref[...])
    nh = _ln(nh, nlng_ref[...], nlnb_ref[...])
    nh = jnp.dot(nh, n1w_ref[...], preferred_element_type=_f32, precision=_HI) + n1b_ref[...]
    h_new = _ln(h + nh, nng_ref[...], nnb_ref[...])
    x_new16 = x16 + ALPHA * x_agg16
    tnew_ref[...] = jnp.concatenate([h_new, x_new16], axis=1)


def _head_body(e_ref, lng_ref, lnb_ref, l0w_ref, l0b_ref,
               l1w_ref, l1b_ref, l2p_ref, l2b_ref, out_ref):
    t = _ln(e_ref[...], lng_ref[...], lnb_ref[...])
    t = _silu(jnp.dot(t, l0w_ref[...], preferred_element_type=_f32, precision=_HI)
              + l0b_ref[...])
    t = _silu(jnp.dot(t, l1w_ref[...], preferred_element_type=_f32, precision=_HI)
              + l1b_ref[...])
    out_ref[...] = (jnp.dot(t, l2p_ref[...], preferred_element_type=_f32,
                            precision=_HI)[:, :1] + l2b_ref[...])


def _row2d(v):
    return v.reshape(1, -1)


def kernel(node_features, edge_features, coords, edge_index, params):
    row = edge_index[0]
    col = edge_index[1]
    nf16 = jnp.pad(node_features, ((0, 0), (0, 16 - node_features.shape[1])))
    ef16 = jnp.pad(edge_features, ((0, 0), (0, 16 - edge_features.shape[1])))
    xc16 = jnp.pad(coords, ((0, 0), (0, 16 - coords.shape[1])))

    ne_w = jnp.pad(params["node_embed"]["w"], ((0, 14), (0, 0)))
    ee_w = jnp.pad(params["edge_embed"]["w"], ((0, 14), (0, 0)))

    n_grid_n = N_NODES // BLK_N
    n_grid_e = N_EDGES // BLK_E

    table = pl.pallas_call(
        _embed_nodes_body,
        grid=(n_grid_n,),
        in_specs=[pl.BlockSpec((BLK_N, 16), lambda i: (i, 0)),
                  pl.BlockSpec((BLK_N, 16), lambda i: (i, 0)),
                  _full((16, NODE_DIM)), _full((1, NODE_DIM))],
        out_specs=pl.BlockSpec((BLK_N, TW), lambda i: (i, 0)),
        out_shape=jax.ShapeDtypeStruct((N_NODES, TW), _f32),
    )(nf16, xc16, ne_w, _row2d(params["node_embed"]["b"]))

    e = pl.pallas_call(
        _embed_edges_body,
        grid=(n_grid_e,),
        in_specs=[pl.BlockSpec((BLK_E, 16), lambda i: (i, 0)),
                  _full((16, EDGE_DIM)), _full((1, EDGE_DIM))],
        out_specs=pl.BlockSpec((BLK_E, EDGE_DIM), lambda i: (i, 0)),
        out_shape=jax.ShapeDtypeStruct((N_EDGES, EDGE_DIM), _f32),
    )(ef16, ee_w, _row2d(params["edge_embed"]["b"]))

    perm = jnp.argsort(row, stable=True).astype(jnp.int32)
    row_sorted = jnp.take(row, perm)
    starts = jnp.asarray([(t // NS) * (N_NODES // NC) + (t % NS) * NPT
                          for t in range(NW)], jnp.int32)
    offs = jnp.searchsorted(row_sorted, starts).astype(jnp.int32)
    offs = jnp.concatenate([offs, jnp.full((1,), N_EDGES, jnp.int32)])
    offs_rep = jnp.broadcast_to(offs[:, None], (NW + 1, 16))
    zeros_acc = jnp.zeros((ACC_ROWS, MW), _f32)

    edge_call = pl.pallas_call(
        _edge_body,
        grid=(n_grid_e,),
        in_specs=[pl.BlockSpec((BLK_E, TW), lambda i: (i, 0)),
                  pl.BlockSpec((BLK_E, TW), lambda i: (i, 0)),
                  pl.BlockSpec((BLK_E, EDGE_DIM), lambda i: (i, 0)),
                  _full((2 * NODE_DIM + EDGE_DIM + 1, HIDDEN)),
                  _full((1, HIDDEN)),
                  _full((1, HIDDEN)), _full((1, HIDDEN)),
                  _full((HIDDEN, HIDDEN)), _full((1, HIDDEN)),
                  _full((HIDDEN, HIDDEN)), _full((1, HIDDEN)),
                  _full((HIDDEN, HIDDEN)), _full((1, HIDDEN)),
                  _full((HIDDEN, HIDDEN)),
                  _full((EDGE_DIM + HIDDEN, HIDDEN)),
                  _full((1, HIDDEN)), _full((1, HIDDEN)), _full((1, HIDDEN)),
                  _full((HIDDEN, EDGE_DIM)), _full((1, EDGE_DIM)),
                  _full((1, EDGE_DIM)), _full((1, EDGE_DIM))],
        out_specs=[pl.BlockSpec((BLK_E, MW), lambda i: (i, 0)),
                   pl.BlockSpec((BLK_E, EDGE_DIM), lambda i: (i, 0))],
        out_shape=[jax.ShapeDtypeStruct((N_EDGES, MW), _f32),
                   jax.ShapeDtypeStruct((N_EDGES, EDGE_DIM), _f32)],
    )

    node_call = pl.pallas_call(
        _node_body,
        grid=(n_grid_n,),
        in_specs=[pl.BlockSpec((BLK_N, TW), lambda i: (i, 0)),
                  pl.BlockSpec((BLK_N, MW), lambda i: (i, 0)),
                  _full((NODE_DIM + HIDDEN, HIDDEN)),
                  _full((1, HIDDEN)), _full((1, HIDDEN)), _full((1, HIDDEN)),
                  _full((HIDDEN, NODE_DIM)), _full((1, NODE_DIM)),
                  _full((1, NODE_DIM)), _full((1, NODE_DIM))],
        out_specs=pl.BlockSpec((BLK_N, TW), lambda i: (i, 0)),
        out_shape=jax.ShapeDtypeStruct((N_NODES, TW), _f32),
    )

    sc_gather, sc_scatter = _sc_kernels()

    for lp in params["layers"]:
        grow, gcol = sc_gather(table, row, col)

        msc, e = edge_call(
            grow, gcol, e,
            lp["m0"]["w"], _row2d(lp["m0"]["b"]),
            _row2d(lp["mln"]["g"]), _row2d(lp["mln"]["b"]),
            lp["m1"]["w"], _row2d(lp["m1"]["b"]),
            lp["m2"]["w"], _row2d(lp["m2"]["b"]),
            lp["c0"]["w"], _row2d(lp["c0"]["b"]),
            jnp.pad(lp["c1"]["w"], ((0, 0), (0, HIDDEN - 1))),
            lp["e0"]["w"], _row2d(lp["e0"]["b"]),
            _row2d(lp["eln"]["g"]), _row2d(lp["eln"]["b"]),
            lp["e1"]["w"], _row2d(lp["e1"]["b"]),
            _row2d(lp["edge_norm"]["g"]), _row2d(lp["edge_norm"]["b"]))

        agg = sc_scatter(msc, perm, row_sorted, offs_rep, zeros_acc)

        table = node_call(
            table, agg,
            lp["n0"]["w"], _row2d(lp["n0"]["b"]),
            _row2d(lp["nln"]["g"]), _row2d(lp["nln"]["b"]),
            lp["n1"]["w"], _row2d(lp["n1"]["b"]),
            _row2d(lp["node_norm"]["g"]), _row2d(lp["node_norm"]["b"]))

    hp = params["head"]
    out = pl.pallas_call(
        _head_body,
        grid=(n_grid_e,),
        in_specs=[pl.BlockSpec((BLK_E, EDGE_DIM), lambda i: (i, 0)),
                  _full((1, EDGE_DIM)), _full((1, EDGE_DIM)),
                  _full((EDGE_DIM, HIDDEN)), _full((1, HIDDEN)),
                  _full((HIDDEN, HIDDEN // 2)), _full((1, HIDDEN // 2)),
                  _full((HIDDEN // 2, HIDDEN)), _full((1, 1))],
        out_specs=pl.BlockSpec((BLK_E, 1), lambda i: (i, 0)),
        out_shape=jax.ShapeDtypeStruct((N_EDGES, 1), _f32),
    )(e, _row2d(hp["ln"]["g"]), _row2d(hp["ln"]["b"]),
      hp["l0"]["w"], _row2d(hp["l0"]["b"]),
      hp["l1"]["w"], _row2d(hp["l1"]["b"]),
      jnp.pad(hp["l2"]["w"], ((0, 0), (0, HIDDEN - 1))),
      _row2d(hp["l2"]["b"]))
    return out[:, 0]
